# baseline (speedup 1.0000x reference)
import functools

import jax
import jax.numpy as jnp
from jax import lax
from jax.experimental import pallas as pl
from jax.experimental.pallas import tpu as pltpu
from jax.experimental.pallas import tpu_sc as plsc

_XMIN = 0.0
_YMIN = 0.0
_RESOLUTION = 0.000244140625
_INV_RES = 1.0 / _RESOLUTION

_GX = 4096
_GY = 4096
_B = 16384
_NC = 2
_NS = 16
_NW = _NC * _NS
_BPW = _B // _NW
_L = 16
_CHUNK = 128
_NCH = _BPW // _CHUNK


def _grid_gather_body(xc_hbm, yc_hbm, gflat_hbm, out_hbm, xv, yv, idxv, resv, sem):
    wid = lax.axis_index("s") * _NC + lax.axis_index("c")
    base = wid * _BPW

    pltpu.sync_copy(xc_hbm.at[pl.ds(base, _BPW)], xv)
    pltpu.sync_copy(yc_hbm.at[pl.ds(base, _BPW)], yv)

    for i in range(_BPW // _L):
        xs = xv[pl.ds(i * _L, _L)]
        ys = yv[pl.ds(i * _L, _L)]
        ix = jnp.minimum(
            jnp.maximum((xs * _INV_RES).astype(jnp.int32), 0), _GX - 1)
        iy = jnp.minimum(
            jnp.maximum((ys * _INV_RES).astype(jnp.int32), 0), _GY - 1)
        flat = ix * _GY + iy
        idxv[i // (_CHUNK // _L), pl.ds((i % (_CHUNK // _L)) * _L, _L)] = flat

    copies = [
        pltpu.make_async_copy(gflat_hbm.at[idxv.at[j]], resv.at[j], sem)
        for j in range(_NCH)
    ]
    for cp in copies:
        cp.start()
    for cp in copies:
        cp.wait()

    for j in range(_NCH):
        pltpu.sync_copy(resv.at[j], out_hbm.at[pl.ds(base + j * _CHUNK, _CHUNK)])


@jax.jit
def kernel(x, grid):
    xc = x[:, 0]
    yc = x[:, 1]
    gflat = grid.reshape(-1)

    mesh = plsc.VectorSubcoreMesh(core_axis_name="c", subcore_axis_name="s")
    run = pl.kernel(
        _grid_gather_body,
        out_type=jax.ShapeDtypeStruct((_B,), jnp.float32),
        mesh=mesh,
        scratch_types=[
            pltpu.VMEM((_BPW,), jnp.float32),
            pltpu.VMEM((_BPW,), jnp.float32),
            pltpu.VMEM((_NCH, _CHUNK), jnp.int32),
            pltpu.VMEM((_NCH, _CHUNK), jnp.float32),
            pltpu.SemaphoreType.DMA,
        ],
    )
    return run(xc, yc, gflat)
